# baseline (speedup 1.0000x reference)
import functools

import jax
import jax.numpy as jnp
from jax.experimental import pallas as pl
from jax.experimental.pallas import tpu as pltpu


def _rvq_body(x_ref, cb_ref, enc_ref, q_ref, sc_ref, split_ref, xaug_ref,
              qbuf_ref, *, n_q, k, d):
    @pl.when(pl.program_id(0) == 0)
    def _():
        cb = cb_ref[...]
        hi = cb.astype(jnp.bfloat16)
        r1 = cb - hi.astype(jnp.float32)
        mid = r1.astype(jnp.bfloat16)
        lo = (r1 - mid.astype(jnp.float32)).astype(jnp.bfloat16)
        iota_k = jax.lax.broadcasted_iota(jnp.int32, (n_q, k, 1), 1)
        ia = ((iota_k >> 6) << 6).astype(jnp.float32).astype(jnp.bfloat16)
        ib = (iota_k & 63).astype(jnp.float32).astype(jnp.bfloat16)
        zpad5 = jnp.zeros((n_q, k, 5), jnp.bfloat16)
        onescol = jnp.ones((n_q, k, 1), jnp.bfloat16)
        split_ref[...] = jnp.concatenate(
            [hi, mid, lo, ia, ib, onescol, zpad5], axis=2)
        nhn = -0.5 * jnp.sum(cb * cb, axis=2, keepdims=True)
        h1 = nhn.astype(jnp.bfloat16)
        s1 = nhn - h1.astype(jnp.float32)
        h2 = s1.astype(jnp.bfloat16)
        h3 = (s1 - h2.astype(jnp.float32)).astype(jnp.bfloat16)
        sc_ref[...] = jnp.concatenate([hi, h1, h2, h3, zpad5], axis=2)
        qbuf_ref[:, d:] = jnp.zeros((qbuf_ref.shape[0], 8), jnp.float32)

    xb = x_ref[...]
    bn = xb.shape[0]
    xaug_ref[...] = jnp.concatenate(
        [xb, jnp.ones((bn, 3), jnp.float32), jnp.zeros((bn, 5), jnp.float32)],
        axis=1)
    n_ch = 2
    hb = bn // n_ch
    ones_row = jnp.ones((1, hb), jnp.bfloat16)
    dnum_t = (((1,), (1,)), ((), ()))
    dnum = (((1,), (0,)), ((), ()))
    cnt_acc = [jnp.zeros((hb, 1), jnp.float32) for _ in range(n_ch)]
    spans = [(h * hb, (h + 1) * hb) for h in range(n_ch)]
    for j in range(n_q):
        scores_h = []
        for lo_, hi_ in spans:
            if j == 0:
                r_aug = xaug_ref[lo_:hi_, :].astype(jnp.bfloat16)
            else:
                r_aug = (xaug_ref[lo_:hi_, :]
                         - qbuf_ref[lo_:hi_, :]).astype(jnp.bfloat16)
            scores_h.append(jax.lax.dot_general(
                r_aug, sc_ref[j], dnum_t,
                preferred_element_type=jnp.float32))
        for h, (lo_, hi_) in enumerate(spans):
            scores = scores_h[h]
            m = jnp.max(scores, axis=1, keepdims=True)
            onehot = jnp.where(scores == m, 1.0, 0.0).astype(jnp.bfloat16)
            g = jax.lax.dot_general(onehot, split_ref[j], dnum,
                                    preferred_element_type=jnp.float32)
            gsum = (g[:, :d] + g[:, d:2 * d]) + g[:, 2 * d:3 * d]
            if j == 0:
                qbuf_ref[lo_:hi_, :d] = gsum
            else:
                qbuf_ref[lo_:hi_, :d] = qbuf_ref[lo_:hi_, :d] + gsum
            ge = g[:, 3 * d:3 * d + 8]
            enc_ref[lo_:hi_, j:j + 1] = (
                ge[:, 0:1] + ge[:, 1:2]).astype(jnp.int32)
            cnt_acc[h] = cnt_acc[h] + ge[:, 2:3]
    q_ref[...] = qbuf_ref[:, :d]
    cnt_all = sum(cnt_acc).astype(jnp.bfloat16)
    tot = jax.lax.dot_general(ones_row, cnt_all, dnum,
                              preferred_element_type=jnp.float32)[0, 0]

    @pl.when(tot > n_q * bn + 0.5)
    def _():
        lane_iota = jax.lax.broadcasted_iota(jnp.int32, (bn, k), 1)
        qe = jnp.zeros((bn, d), jnp.float32)
        for j in range(n_q):
            r_aug = jnp.concatenate(
                [(xb - qe), jnp.ones((bn, 3), jnp.float32),
                 jnp.zeros((bn, 5), jnp.float32)], axis=1)
            scores = jax.lax.dot_general(
                r_aug.astype(jnp.bfloat16), sc_ref[j], dnum_t,
                preferred_element_type=jnp.float32)
            m = jnp.max(scores, axis=1, keepdims=True)
            idx = jnp.min(jnp.where(scores == m, lane_iota, k), axis=1)
            enc_ref[:, j:j + 1] = idx[:, None]
            onehot = (lane_iota == idx[:, None]).astype(
                jnp.float32).astype(jnp.bfloat16)
            g = jax.lax.dot_general(onehot, split_ref[j], dnum,
                                    preferred_element_type=jnp.float32)
            qe = qe + ((g[:, :d] + g[:, d:2 * d]) + g[:, 2 * d:3 * d])
        q_ref[...] = qe


@jax.jit
def kernel(x, codebooks):
    n, d = x.shape
    n_q, k, _ = codebooks.shape
    bn = 1024
    enc, quant = pl.pallas_call(
        functools.partial(_rvq_body, n_q=n_q, k=k, d=d),
        grid=(n // bn,),
        in_specs=[
            pl.BlockSpec((bn, d), lambda i: (i, 0)),
            pl.BlockSpec((n_q, k, d), lambda i: (0, 0, 0)),
        ],
        out_specs=[
            pl.BlockSpec((bn, n_q), lambda i: (i, 0)),
            pl.BlockSpec((bn, d), lambda i: (i, 0)),
        ],
        out_shape=[
            jax.ShapeDtypeStruct((n, n_q), jnp.int32),
            jax.ShapeDtypeStruct((n, d), jnp.float32),
        ],
        scratch_shapes=[
            pltpu.VMEM((n_q, k, d + 8), jnp.bfloat16),
            pltpu.VMEM((n_q, k, 3 * d + 8), jnp.bfloat16),
            pltpu.VMEM((bn, d + 8), jnp.float32),
            pltpu.VMEM((bn, d + 8), jnp.float32),
        ],
    )(x, codebooks)
    return (enc, quant)


# speedup vs baseline: 1.0060x; 1.0060x over previous
import functools

import jax
import jax.numpy as jnp
from jax.experimental import pallas as pl
from jax.experimental.pallas import tpu as pltpu


def _rvq_body(x_ref, cb_ref, enc_ref, q_ref, sc_ref, split_ref, xaug_ref,
              qbuf_ref, *, n_q, k, d):
    @pl.when(pl.program_id(0) == 0)
    def _():
        cb = cb_ref[...]
        hi = cb.astype(jnp.bfloat16)
        r1 = cb - hi.astype(jnp.float32)
        mid = r1.astype(jnp.bfloat16)
        lo = (r1 - mid.astype(jnp.float32)).astype(jnp.bfloat16)
        iota_k = jax.lax.broadcasted_iota(jnp.int32, (n_q, k, 1), 1)
        ia = ((iota_k >> 6) << 6).astype(jnp.float32).astype(jnp.bfloat16)
        ib = (iota_k & 63).astype(jnp.float32).astype(jnp.bfloat16)
        zpad5 = jnp.zeros((n_q, k, 5), jnp.bfloat16)
        onescol = jnp.ones((n_q, k, 1), jnp.bfloat16)
        split_ref[...] = jnp.concatenate(
            [hi, mid, lo, ia, ib, onescol, zpad5], axis=2)
        nhn = -0.5 * jnp.sum(cb * cb, axis=2, keepdims=True)
        h1 = nhn.astype(jnp.bfloat16)
        s1 = nhn - h1.astype(jnp.float32)
        h2 = s1.astype(jnp.bfloat16)
        h3 = (s1 - h2.astype(jnp.float32)).astype(jnp.bfloat16)
        sc_ref[...] = jnp.concatenate([hi, h1, h2, h3, zpad5], axis=2)
        qbuf_ref[:, d:] = jnp.zeros((qbuf_ref.shape[0], 8), jnp.float32)

    xb = x_ref[...]
    bn = xb.shape[0]
    xaug_ref[...] = jnp.concatenate(
        [xb, jnp.ones((bn, 3), jnp.float32), jnp.zeros((bn, 5), jnp.float32)],
        axis=1)
    n_ch = 2
    hb = bn // n_ch
    ones_row = jnp.ones((1, hb), jnp.bfloat16)
    dnum_t = (((1,), (1,)), ((), ()))
    dnum = (((1,), (0,)), ((), ()))
    cnt_acc = [jnp.zeros((hb, 1), jnp.float32) for _ in range(n_ch)]
    spans = [(h * hb, (h + 1) * hb) for h in range(n_ch)]
    for j in range(n_q):
        scores_h = []
        for lo_, hi_ in spans:
            if j == 0:
                r_aug = xaug_ref[lo_:hi_, :].astype(jnp.bfloat16)
            else:
                r_aug = (xaug_ref[lo_:hi_, :]
                         - qbuf_ref[lo_:hi_, :]).astype(jnp.bfloat16)
            scores_h.append(jax.lax.dot_general(
                r_aug, sc_ref[j], dnum_t,
                preferred_element_type=jnp.float32))
        onehot_h = []
        for scores in scores_h:
            m = jnp.max(scores, axis=1, keepdims=True)
            onehot_h.append(
                jnp.where(scores == m, 1.0, 0.0).astype(jnp.bfloat16))
        for h, (lo_, hi_) in enumerate(spans):
            onehot = onehot_h[h]
            g = jax.lax.dot_general(onehot, split_ref[j], dnum,
                                    preferred_element_type=jnp.float32)
            gsum = (g[:, :d] + g[:, d:2 * d]) + g[:, 2 * d:3 * d]
            if j == 0:
                qbuf_ref[lo_:hi_, :d] = gsum
            else:
                qbuf_ref[lo_:hi_, :d] = qbuf_ref[lo_:hi_, :d] + gsum
            ge = g[:, 3 * d:3 * d + 8]
            enc_ref[lo_:hi_, j:j + 1] = (
                ge[:, 0:1] + ge[:, 1:2]).astype(jnp.int32)
            cnt_acc[h] = cnt_acc[h] + ge[:, 2:3]
    q_ref[...] = qbuf_ref[:, :d]
    cnt_all = sum(cnt_acc).astype(jnp.bfloat16)
    tot = jax.lax.dot_general(ones_row, cnt_all, dnum,
                              preferred_element_type=jnp.float32)[0, 0]

    @pl.when(tot > n_q * bn + 0.5)
    def _():
        lane_iota = jax.lax.broadcasted_iota(jnp.int32, (bn, k), 1)
        qe = jnp.zeros((bn, d), jnp.float32)
        for j in range(n_q):
            r_aug = jnp.concatenate(
                [(xb - qe), jnp.ones((bn, 3), jnp.float32),
                 jnp.zeros((bn, 5), jnp.float32)], axis=1)
            scores = jax.lax.dot_general(
                r_aug.astype(jnp.bfloat16), sc_ref[j], dnum_t,
                preferred_element_type=jnp.float32)
            m = jnp.max(scores, axis=1, keepdims=True)
            idx = jnp.min(jnp.where(scores == m, lane_iota, k), axis=1)
            enc_ref[:, j:j + 1] = idx[:, None]
            onehot = (lane_iota == idx[:, None]).astype(
                jnp.float32).astype(jnp.bfloat16)
            g = jax.lax.dot_general(onehot, split_ref[j], dnum,
                                    preferred_element_type=jnp.float32)
            qe = qe + ((g[:, :d] + g[:, d:2 * d]) + g[:, 2 * d:3 * d])
        q_ref[...] = qe


@jax.jit
def kernel(x, codebooks):
    n, d = x.shape
    n_q, k, _ = codebooks.shape
    bn = 1024
    enc, quant = pl.pallas_call(
        functools.partial(_rvq_body, n_q=n_q, k=k, d=d),
        grid=(n // bn,),
        in_specs=[
            pl.BlockSpec((bn, d), lambda i: (i, 0)),
            pl.BlockSpec((n_q, k, d), lambda i: (0, 0, 0)),
        ],
        out_specs=[
            pl.BlockSpec((bn, n_q), lambda i: (i, 0)),
            pl.BlockSpec((bn, d), lambda i: (i, 0)),
        ],
        out_shape=[
            jax.ShapeDtypeStruct((n, n_q), jnp.int32),
            jax.ShapeDtypeStruct((n, d), jnp.float32),
        ],
        scratch_shapes=[
            pltpu.VMEM((n_q, k, d + 8), jnp.bfloat16),
            pltpu.VMEM((n_q, k, 3 * d + 8), jnp.bfloat16),
            pltpu.VMEM((bn, d + 8), jnp.float32),
            pltpu.VMEM((bn, d + 8), jnp.float32),
        ],
    )(x, codebooks)
    return (enc, quant)
